# baseline (speedup 1.0000x reference)
import functools

import jax
import jax.numpy as jnp
from jax import lax
from jax.experimental import pallas as pl
from jax.experimental.pallas import tpu as pltpu


def _burst(fq_hbm, fk_hbm, out_ref, buf, sem, *, nchunks, rows, nbuf):
    for c in range(nchunks):
        src = fq_hbm if c % 2 == 0 else fk_hbm
        pltpu.make_async_copy(
            src.at[pl.ds((c // 2) * rows, rows)], buf.at[c % nbuf], sem.at[c % nbuf]
        ).start()
    for c in range(nchunks):
        src = fq_hbm if c % 2 == 0 else fk_hbm
        pltpu.make_async_copy(
            src.at[pl.ds((c // 2) * rows, rows)], buf.at[c % nbuf], sem.at[c % nbuf]
        ).wait()
    out_ref[...] = jnp.sum(buf[0, 0, 0:8, 0:128], axis=0, keepdims=True).reshape(1, 128)


def kernel(features_q, features_k, mask):
    M, B, C, H, W = features_q.shape
    N = M * B
    HW = H * W
    fq = features_q.reshape(N, C, HW)
    fk = features_k.reshape(N, C, HW)
    rows = 4
    nchunks = 2 * (N // rows)
    nbuf = 8

    out_row = pl.pallas_call(
        functools.partial(_burst, nchunks=nchunks, rows=rows, nbuf=nbuf),
        in_specs=[
            pl.BlockSpec(memory_space=pl.ANY),
            pl.BlockSpec(memory_space=pl.ANY),
        ],
        out_specs=pl.BlockSpec(memory_space=pltpu.MemorySpace.VMEM),
        out_shape=jax.ShapeDtypeStruct((1, 128), jnp.float32),
        scratch_shapes=[
            pltpu.VMEM((nbuf, rows, C, HW), jnp.float32),
            pltpu.SemaphoreType.DMA((nbuf,)),
        ],
        compiler_params=pltpu.CompilerParams(
            vmem_limit_bytes=48 * 1024 * 1024),
    )(fq, fk)

    loss = out_row[0, 0]
    loss_dict = {'loss': loss,
                 'pos_cos_sim': loss,
                 'neg_cos_sim': loss,
                 'pos_softmax_sim': loss,
                 'neg_softmax_sim': loss}
    return loss, loss_dict


# speedup vs baseline: 2.7833x; 2.7833x over previous
import functools

import jax
import jax.numpy as jnp
from jax import lax
from jax.experimental import pallas as pl
from jax.experimental.pallas import tpu as pltpu


def _epilogue(pooled_q, pooled_k, counts, *, inv_tau, n):
    cnt = jnp.maximum(counts, 1.0)
    mq = pooled_q / cnt
    mk = pooled_k / cnt

    padf = (mk[:, 0:1] != 0.0).astype(jnp.float32)

    eps2 = jnp.float32(1e-24)
    qn = mq * lax.rsqrt(jnp.maximum(jnp.sum(mq * mq, -1, keepdims=True), eps2))
    kn = mk * lax.rsqrt(jnp.maximum(jnp.sum(mk * mk, -1, keepdims=True), eps2))

    sim = lax.dot_general(kn, qn, (((1,), (1,)), ((), ())),
                          preferred_element_type=jnp.float32)

    ridx = lax.broadcasted_iota(jnp.int32, (n, n), 0)
    cidx = lax.broadcasted_iota(jnp.int32, (n, n), 1)
    diagf = (ridx == cidx).astype(jnp.float32)

    logits = sim * jnp.float32(inv_tau)
    row_max = jnp.max(logits, axis=-1, keepdims=True)
    lse = jnp.log(jnp.sum(jnp.exp(logits - row_max), -1, keepdims=True)) + row_max
    ce = lse - jnp.sum(logits * diagf, axis=-1, keepdims=True)
    loss = jnp.sum(ce * padf) / jnp.sum(padf)

    nf = jnp.float32(n)
    diag_sum = jnp.sum(sim * diagf)
    pos_cos = diag_sum / nf
    neg_cos = (jnp.sum(sim) - diag_sum) / (nf * (nf - 1.0))

    s_max = jnp.max(sim, axis=-1, keepdims=True)
    e = jnp.exp(sim - s_max)
    sm = e / jnp.sum(e, axis=-1, keepdims=True)
    diag_sum_s = jnp.sum(sm * diagf)
    pos_sm = diag_sum_s / nf
    neg_sm = (jnp.sum(sm) - diag_sum_s) / (nf * (nf - 1.0))

    lane = lax.broadcasted_iota(jnp.int32, (1, 128), 1)
    vals = (loss, pos_cos, neg_cos, pos_sm, neg_sm)
    row = jnp.zeros((1, 128), jnp.float32)
    for slot, v in enumerate(vals):
        row = row + jnp.where(lane == slot, v, jnp.float32(0.0))
    return row


def _mono_kernel(fq_hbm, fk_hbm, pos_ref, sel_ref, out_ref,
                 bufq, bufk, pq, pk, semq, semk,
                 *, nchunks, rows, depth, inv_tau, n):
    def q_copy(c, slot):
        return pltpu.make_async_copy(
            fq_hbm.at[pl.ds(c * rows, rows)], bufq.at[slot], semq.at[slot])

    def k_copy(c, slot):
        return pltpu.make_async_copy(
            fk_hbm.at[pl.ds(c * rows, rows)], bufk.at[slot], semk.at[slot])

    for j in range(min(depth, nchunks)):
        q_copy(j, j).start(priority=0)
        k_copy(j, j).start(priority=1)

    w_all = pos_ref[...] * sel_ref[...]

    dims = (((1,), (1,)), ((0,), (0,)))
    for c in range(nchunks):
        slot = c % depth
        w = lax.slice(w_all, (c * rows, 0), (c * rows + rows, w_all.shape[1]))
        nxt = c + depth
        q_copy(c, slot).wait()
        pq[pl.ds(c * rows, rows)] = lax.dot_general(
            bufq[slot], w, dims, preferred_element_type=jnp.float32)
        if nxt < nchunks:
            q_copy(nxt, slot).start(priority=0)
        k_copy(c, slot).wait()
        pk[pl.ds(c * rows, rows)] = lax.dot_general(
            bufk[slot], w, dims, preferred_element_type=jnp.float32)
        if nxt < nchunks:
            k_copy(nxt, slot).start(priority=1)

    counts = jnp.sum(sel_ref[...], axis=-1, keepdims=True)
    out_ref[...] = _epilogue(pq[...], pk[...], counts, inv_tau=inv_tau, n=n)


def kernel(features_q, features_k, mask):
    M, B, C, H, W = features_q.shape
    N = M * B
    HW = H * W

    fq = features_q.transpose(0, 1, 3, 4, 2).reshape(N, HW, C)
    fk = features_k.transpose(0, 1, 3, 4, 2).reshape(N, HW, C)
    posm = jnp.transpose(mask, (1, 0, 2, 3)).reshape(N, HW).astype(jnp.float32)
    selm = mask.reshape(N, HW).astype(jnp.float32)

    if N % 4 == 0:
        rows = 4
    else:
        rows = N
    nchunks = N // rows
    depth = min(6, nchunks)

    out_row = pl.pallas_call(
        functools.partial(_mono_kernel, nchunks=nchunks, rows=rows,
                          depth=depth, inv_tau=1.0 / 0.1, n=N),
        in_specs=[
            pl.BlockSpec(memory_space=pl.ANY),
            pl.BlockSpec(memory_space=pl.ANY),
            pl.BlockSpec(memory_space=pltpu.MemorySpace.VMEM),
            pl.BlockSpec(memory_space=pltpu.MemorySpace.VMEM),
        ],
        out_specs=pl.BlockSpec(memory_space=pltpu.MemorySpace.VMEM),
        out_shape=jax.ShapeDtypeStruct((1, 128), jnp.float32),
        scratch_shapes=[
            pltpu.VMEM((depth, rows, HW, C), jnp.float32),
            pltpu.VMEM((depth, rows, HW, C), jnp.float32),
            pltpu.VMEM((N, C), jnp.float32),
            pltpu.VMEM((N, C), jnp.float32),
            pltpu.SemaphoreType.DMA((depth,)),
            pltpu.SemaphoreType.DMA((depth,)),
        ],
        compiler_params=pltpu.CompilerParams(
            vmem_limit_bytes=48 * 1024 * 1024),
    )(fq, fk, posm, selm)

    loss = out_row[0, 0]
    loss_dict = {'loss': loss,
                 'pos_cos_sim': out_row[0, 1],
                 'neg_cos_sim': out_row[0, 2],
                 'pos_softmax_sim': out_row[0, 3],
                 'neg_softmax_sim': out_row[0, 4]}
    return loss, loss_dict


# speedup vs baseline: 3.3049x; 1.1874x over previous
import functools

import jax
import jax.numpy as jnp
from jax import lax
from jax.experimental import pallas as pl
from jax.experimental.pallas import tpu as pltpu


def _epilogue(pooled_q, pooled_k, counts, *, inv_tau, n):
    cnt = jnp.maximum(counts, 1.0)
    mq = pooled_q / cnt
    mk = pooled_k / cnt

    padf = (mk[:, 0:1] != 0.0).astype(jnp.float32)

    eps2 = jnp.float32(1e-24)
    qn = mq * lax.rsqrt(jnp.maximum(jnp.sum(mq * mq, -1, keepdims=True), eps2))
    kn = mk * lax.rsqrt(jnp.maximum(jnp.sum(mk * mk, -1, keepdims=True), eps2))

    sim = lax.dot_general(kn, qn, (((1,), (1,)), ((), ())),
                          preferred_element_type=jnp.float32)

    ridx = lax.broadcasted_iota(jnp.int32, (n, n), 0)
    cidx = lax.broadcasted_iota(jnp.int32, (n, n), 1)
    diagf = (ridx == cidx).astype(jnp.float32)

    logits = sim * jnp.float32(inv_tau)
    row_max = jnp.max(logits, axis=-1, keepdims=True)
    lse = jnp.log(jnp.sum(jnp.exp(logits - row_max), -1, keepdims=True)) + row_max
    ce = lse - jnp.sum(logits * diagf, axis=-1, keepdims=True)
    loss = jnp.sum(ce * padf) / jnp.sum(padf)

    nf = jnp.float32(n)
    diag_sum = jnp.sum(sim * diagf)
    pos_cos = diag_sum / nf
    neg_cos = (jnp.sum(sim) - diag_sum) / (nf * (nf - 1.0))

    s_max = jnp.max(sim, axis=-1, keepdims=True)
    e = jnp.exp(sim - s_max)
    sm = e / jnp.sum(e, axis=-1, keepdims=True)
    diag_sum_s = jnp.sum(sm * diagf)
    pos_sm = diag_sum_s / nf
    neg_sm = (jnp.sum(sm) - diag_sum_s) / (nf * (nf - 1.0))

    lane = lax.broadcasted_iota(jnp.int32, (1, 128), 1)
    vals = (loss, pos_cos, neg_cos, pos_sm, neg_sm)
    row = jnp.zeros((1, 128), jnp.float32)
    for slot, v in enumerate(vals):
        row = row + jnp.where(lane == slot, v, jnp.float32(0.0))
    return row


def _mono_kernel(fq_hbm, fk_hbm, pos_ref, sel_ref, out_ref,
                 bufq, bufk, pq, pk, semq, semk,
                 *, nchunks, rows, depth, inv_tau, n):
    def q_copy(c, slot):
        return pltpu.make_async_copy(
            fq_hbm.at[pl.ds(c * rows, rows)], bufq.at[slot], semq.at[slot])

    def k_copy(c, slot):
        return pltpu.make_async_copy(
            fk_hbm.at[pl.ds(c * rows, rows)], bufk.at[slot], semk.at[slot])

    for j in range(min(depth, nchunks)):
        q_copy(j, j).start(priority=0)
        k_copy(j, j).start(priority=1)

    w_all = pos_ref[...] * sel_ref[...]

    for c in range(nchunks):
        slot = c % depth
        w = lax.slice(w_all, (c * rows, 0), (c * rows + rows, w_all.shape[1]))
        wb = w[:, :, None]
        nxt = c + depth
        q_copy(c, slot).wait()
        pq[pl.ds(c * rows, rows)] = jnp.sum(bufq[slot] * wb, axis=1)
        if nxt < nchunks:
            q_copy(nxt, slot).start(priority=0)
        k_copy(c, slot).wait()
        pk[pl.ds(c * rows, rows)] = jnp.sum(bufk[slot] * wb, axis=1)
        if nxt < nchunks:
            k_copy(nxt, slot).start(priority=1)

    counts = jnp.sum(sel_ref[...], axis=-1, keepdims=True)
    out_ref[...] = _epilogue(pq[...], pk[...], counts, inv_tau=inv_tau, n=n)


def kernel(features_q, features_k, mask):
    M, B, C, H, W = features_q.shape
    N = M * B
    HW = H * W

    fq = features_q.transpose(0, 1, 3, 4, 2).reshape(N, HW, C)
    fk = features_k.transpose(0, 1, 3, 4, 2).reshape(N, HW, C)
    posm = jnp.transpose(mask, (1, 0, 2, 3)).reshape(N, HW).astype(jnp.float32)
    selm = mask.reshape(N, HW).astype(jnp.float32)

    if N % 4 == 0:
        rows = 4
    else:
        rows = N
    nchunks = N // rows
    depth = min(6, nchunks)

    out_row = pl.pallas_call(
        functools.partial(_mono_kernel, nchunks=nchunks, rows=rows,
                          depth=depth, inv_tau=1.0 / 0.1, n=N),
        in_specs=[
            pl.BlockSpec(memory_space=pl.ANY),
            pl.BlockSpec(memory_space=pl.ANY),
            pl.BlockSpec(memory_space=pltpu.MemorySpace.VMEM),
            pl.BlockSpec(memory_space=pltpu.MemorySpace.VMEM),
        ],
        out_specs=pl.BlockSpec(memory_space=pltpu.MemorySpace.VMEM),
        out_shape=jax.ShapeDtypeStruct((1, 128), jnp.float32),
        scratch_shapes=[
            pltpu.VMEM((depth, rows, HW, C), jnp.float32),
            pltpu.VMEM((depth, rows, HW, C), jnp.float32),
            pltpu.VMEM((N, C), jnp.float32),
            pltpu.VMEM((N, C), jnp.float32),
            pltpu.SemaphoreType.DMA((depth,)),
            pltpu.SemaphoreType.DMA((depth,)),
        ],
        compiler_params=pltpu.CompilerParams(
            vmem_limit_bytes=48 * 1024 * 1024),
    )(fq, fk, posm, selm)

    loss = out_row[0, 0]
    loss_dict = {'loss': loss,
                 'pos_cos_sim': out_row[0, 1],
                 'neg_cos_sim': out_row[0, 2],
                 'pos_softmax_sim': out_row[0, 3],
                 'neg_softmax_sim': out_row[0, 4]}
    return loss, loss_dict
